# baseline (speedup 1.0000x reference)
import functools

import jax
import jax.numpy as jnp
from jax import lax
from jax.experimental import pallas as pl
from jax.experimental.pallas import tpu as pltpu
from jax.experimental.pallas import tpu_sc as plsc

LANES = 16
NC = 2
NS = 16
NW = NC * NS
N_, C_, H_, W_ = 4, 96, 224, 224
ROWS = N_ * C_
ROWS_PER_W = ROWS // NW
BH = 112
BLOCKS_PER_SLAB = H_ // BH
NBLOCK = ROWS_PER_W * BLOCKS_PER_SLAB
WREGS = W_ // LANES
TAB = 64
BIASF = 16384.0
BIASI = 16384

_mesh = plsc.VectorSubcoreMesh(core_axis_name="c", subcore_axis_name="s")


@functools.partial(
    pl.kernel,
    mesh=_mesh,
    compiler_params=pltpu.CompilerParams(
        needs_layout_passes=False, use_tc_tiling_on_sc=True
    ),
    out_type=jax.ShapeDtypeStruct((ROWS, H_, W_), jnp.float32),
    scratch_types=[
        pltpu.VMEM((C_ * TAB,), jnp.float32),
        pltpu.VMEM((C_ * TAB,), jnp.float32),
        pltpu.VMEM((LANES,), jnp.float32),
        pltpu.VMEM((BH, W_), jnp.float32),
        pltpu.VMEM((BH, W_), jnp.float32),
        pltpu.VMEM((BH, W_), jnp.float32),
        pltpu.VMEM((BH, W_), jnp.float32),
        pltpu.SemaphoreType.DMA,
        pltpu.SemaphoreType.DMA,
        pltpu.SemaphoreType.DMA,
        pltpu.SemaphoreType.DMA,
    ],
)
def _sc_pwl(x_hbm, a_hbm, b_hbm, prm_hbm, out_hbm,
            a_v, b_v, prm_v, in0, in1, out0, out1,
            si0, si1, so0, so1):
    cid = lax.axis_index("c")
    sid = lax.axis_index("s")
    wid = sid * NC + cid
    slab0 = wid * ROWS_PER_W

    in_bufs = (in0, in1)
    out_bufs = (out0, out1)
    in_sems = (si0, si1)
    out_sems = (so0, so1)

    pltpu.sync_copy(a_hbm, a_v)
    pltpu.sync_copy(b_hbm, b_v)
    pltpu.sync_copy(prm_hbm, prm_v)
    pv = prm_v[pl.ds(0, LANES)]
    inv_step = pv[0]
    toff = pv[1]

    def start_in(j, b):
        slab = slab0 + j // BLOCKS_PER_SLAB
        h0 = (j % BLOCKS_PER_SLAB) * BH
        return pltpu.async_copy(
            x_hbm.at[slab, pl.ds(h0, BH), :], in_bufs[b], in_sems[b])

    def wait_in(b):
        pltpu.make_async_copy(
            x_hbm.at[0, pl.ds(0, BH), :], in_bufs[b], in_sems[b]).wait()

    def start_out(j, b):
        slab = slab0 + j // BLOCKS_PER_SLAB
        h0 = (j % BLOCKS_PER_SLAB) * BH
        return pltpu.async_copy(
            out_bufs[b], out_hbm.at[slab, pl.ds(h0, BH), :], out_sems[b])

    def wait_out(b):
        pltpu.make_async_copy(
            out_bufs[b], out_hbm.at[0, pl.ds(0, BH), :], out_sems[b]).wait()

    start_in(0, 0)
    start_in(1, 1)

    def pair_body(jj, carry):
        for b in range(2):
            j = jj * 2 + b
            in_buf = in_bufs[b]
            out_buf = out_bufs[b]

            slab = slab0 + j // BLOCKS_PER_SLAB
            koff = lax.rem(slab, C_) * TAB + 1

            wait_in(b)

            @pl.when(jj > 0)
            def _():
                wait_out(b)

            @plsc.parallel_loop(0, BH, 1, unroll=1)
            def _(h):
                for w in range(WREGS):
                    x = in_buf[h, pl.ds(w * LANES, LANES)]
                    t = x * inv_step + toff
                    ti = t.astype(jnp.int32)
                    tf = ti.astype(jnp.float32)
                    kfl = jnp.where(tf > t, ti - 1, ti)
                    ki = jnp.clip(kfl, -1, 62) + koff
                    a = plsc.load_gather(a_v, [ki])
                    bb = plsc.load_gather(b_v, [ki])
                    out_buf[h, pl.ds(w * LANES, LANES)] = a * x + bb

            start_out(j, b)

            @pl.when(j + 2 < NBLOCK)
            def _():
                start_in(j + 2, b)

        return carry

    lax.fori_loop(0, NBLOCK // 2, pair_body, 0)
    wait_out(0)
    wait_out(1)


def kernel(input, ps, qs):
    qs = qs.astype(jnp.float32)
    ps = ps.astype(jnp.float32)
    step = ps[1] - ps[0]
    inv_step = 1.0 / step
    s_mid = (qs[1:] - qs[:-1]) * inv_step
    b_mid = qs[:-1] - s_mid * ps[:-1, None]
    ones = jnp.ones((1, C_), jnp.float32)
    a_tab = jnp.concatenate([ones, s_mid, ones], axis=0)
    b_tab = jnp.concatenate(
        [(qs[0] - ps[0])[None, :], b_mid, (qs[-1] - ps[-1])[None, :]], axis=0
    )
    a_flat = a_tab.T.reshape(-1)
    b_flat = b_tab.T.reshape(-1)
    prm = jnp.zeros((LANES,), jnp.float32)
    prm = prm.at[0].set(inv_step).at[1].set(-ps[0] * inv_step)

    out3 = _sc_pwl(input.reshape(ROWS, H_, W_), a_flat, b_flat, prm)
    return out3.reshape(input.shape)


# speedup vs baseline: 1.2673x; 1.2673x over previous
import functools

import jax
import jax.numpy as jnp
from jax import lax
from jax.experimental import pallas as pl
from jax.experimental.pallas import tpu as pltpu
from jax.experimental.pallas import tpu_sc as plsc

LANES = 16
NC = 2
NS = 16
NW = NC * NS
N_, C_, H_, W_ = 4, 96, 224, 224
ROWS = N_ * C_
ROWS_PER_W = ROWS // NW
BH = 112
BLOCKS_PER_SLAB = H_ // BH
NBLOCK = ROWS_PER_W * BLOCKS_PER_SLAB
WREGS = W_ // LANES
TAB = 64

_mesh = plsc.VectorSubcoreMesh(core_axis_name="c", subcore_axis_name="s")


@functools.partial(
    pl.kernel,
    mesh=_mesh,
    compiler_params=pltpu.CompilerParams(
        needs_layout_passes=False, use_tc_tiling_on_sc=True
    ),
    out_type=jax.ShapeDtypeStruct((ROWS, H_, W_), jnp.float32),
    scratch_types=[
        pltpu.VMEM((C_ * TAB,), jnp.float32),
        pltpu.VMEM((C_ * TAB,), jnp.float32),
        pltpu.VMEM((LANES,), jnp.float32),
        pltpu.VMEM((BH, W_), jnp.float32),
        pltpu.VMEM((BH, W_), jnp.float32),
        pltpu.VMEM((BH, W_), jnp.float32),
        pltpu.VMEM((BH, W_), jnp.float32),
        pltpu.SemaphoreType.DMA,
        pltpu.SemaphoreType.DMA,
        pltpu.SemaphoreType.DMA,
        pltpu.SemaphoreType.DMA,
    ],
)
def _sc_pwl(x_hbm, a_hbm, b_hbm, prm_hbm, out_hbm,
            a_v, b_v, prm_v, in0, in1, out0, out1,
            si0, si1, so0, so1):
    cid = lax.axis_index("c")
    sid = lax.axis_index("s")
    wid = sid * NC + cid
    slab0 = wid * ROWS_PER_W

    in_bufs = (in0, in1)
    out_bufs = (out0, out1)
    in_sems = (si0, si1)
    out_sems = (so0, so1)

    pltpu.sync_copy(a_hbm, a_v)
    pltpu.sync_copy(b_hbm, b_v)
    pltpu.sync_copy(prm_hbm, prm_v)
    pv = prm_v[pl.ds(0, LANES)]
    inv_step = pv[0]
    toff = pv[1]

    def start_in(j, b):
        slab = slab0 + j // BLOCKS_PER_SLAB
        h0 = (j % BLOCKS_PER_SLAB) * BH
        return pltpu.async_copy(
            x_hbm.at[slab, pl.ds(h0, BH), :], in_bufs[b], in_sems[b])

    def wait_in(b):
        pltpu.make_async_copy(
            x_hbm.at[0, pl.ds(0, BH), :], in_bufs[b], in_sems[b]).wait()

    def start_out(j, b):
        slab = slab0 + j // BLOCKS_PER_SLAB
        h0 = (j % BLOCKS_PER_SLAB) * BH
        return pltpu.async_copy(
            out_bufs[b], out_hbm.at[slab, pl.ds(h0, BH), :], out_sems[b])

    def wait_out(b):
        pltpu.make_async_copy(
            out_bufs[b], out_hbm.at[0, pl.ds(0, BH), :], out_sems[b]).wait()

    start_in(0, 0)
    start_in(1, 1)

    def pair_body(jj, carry):
        for b in range(2):
            j = jj * 2 + b
            in_buf = in_bufs[b]
            out_buf = out_bufs[b]

            slab = slab0 + j // BLOCKS_PER_SLAB
            clo = lax.rem(slab, C_) * TAB
            chi = clo + (TAB - 1)
            boff = toff + (clo + 1).astype(jnp.float32)

            wait_in(b)

            @pl.when(jj > 0)
            def _():
                wait_out(b)

            @plsc.parallel_loop(0, BH, 1, unroll=1)
            def _(h):
                for w in range(WREGS):
                    x = in_buf[h, pl.ds(w * LANES, LANES)]
                    t = x * inv_step + boff
                    ti = t.astype(jnp.int32)
                    ki = jnp.clip(ti, clo, chi)
                    a = plsc.load_gather(a_v, [ki])
                    bb = plsc.load_gather(b_v, [ki])
                    out_buf[h, pl.ds(w * LANES, LANES)] = a * x + bb

            start_out(j, b)

            @pl.when(j + 2 < NBLOCK)
            def _():
                start_in(j + 2, b)

        return carry

    lax.fori_loop(0, NBLOCK // 2, pair_body, 0)
    wait_out(0)
    wait_out(1)


def kernel(input, ps, qs):
    qs = qs.astype(jnp.float32)
    ps = ps.astype(jnp.float32)
    step = ps[1] - ps[0]
    inv_step = 1.0 / step
    s_mid = (qs[1:] - qs[:-1]) * inv_step
    b_mid = qs[:-1] - s_mid * ps[:-1, None]
    ones = jnp.ones((1, C_), jnp.float32)
    a_tab = jnp.concatenate([ones, s_mid, ones], axis=0)
    b_tab = jnp.concatenate(
        [(qs[0] - ps[0])[None, :], b_mid, (qs[-1] - ps[-1])[None, :]], axis=0
    )
    a_flat = a_tab.T.reshape(-1)
    b_flat = b_tab.T.reshape(-1)
    prm = jnp.zeros((LANES,), jnp.float32)
    prm = prm.at[0].set(inv_step).at[1].set(-ps[0] * inv_step)

    out3 = _sc_pwl(input.reshape(ROWS, H_, W_), a_flat, b_flat, prm)
    return out3.reshape(input.shape)


# speedup vs baseline: 1.2860x; 1.0148x over previous
import functools

import jax
import jax.numpy as jnp
from jax import lax
from jax.experimental import pallas as pl
from jax.experimental.pallas import tpu as pltpu
from jax.experimental.pallas import tpu_sc as plsc

LANES = 16
NC = 2
NS = 16
NW = NC * NS
N_, C_, H_, W_ = 4, 96, 224, 224
ROWS = N_ * C_
ROWS_PER_W = ROWS // NW
BH = 112
BLOCKS_PER_SLAB = H_ // BH
NBLOCK = ROWS_PER_W * BLOCKS_PER_SLAB
WREGS = W_ // LANES
TAB = 64

_mesh = plsc.VectorSubcoreMesh(core_axis_name="c", subcore_axis_name="s")


@functools.partial(
    pl.kernel,
    mesh=_mesh,
    compiler_params=pltpu.CompilerParams(
        needs_layout_passes=False, use_tc_tiling_on_sc=True
    ),
    out_type=jax.ShapeDtypeStruct((ROWS, H_, W_), jnp.float32),
    scratch_types=[
        pltpu.VMEM((C_ * TAB,), jnp.float32),
        pltpu.VMEM((C_ * TAB,), jnp.float32),
        pltpu.VMEM((LANES,), jnp.float32),
        pltpu.VMEM((BH, W_), jnp.float32),
        pltpu.VMEM((BH, W_), jnp.float32),
        pltpu.VMEM((BH, W_), jnp.float32),
        pltpu.VMEM((BH, W_), jnp.float32),
        pltpu.SemaphoreType.DMA,
        pltpu.SemaphoreType.DMA,
        pltpu.SemaphoreType.DMA,
        pltpu.SemaphoreType.DMA,
    ],
)
def _sc_pwl(x_hbm, a_hbm, b_hbm, prm_hbm, out_hbm,
            a_v, b_v, prm_v, in0, in1, out0, out1,
            si0, si1, so0, so1):
    cid = lax.axis_index("c")
    sid = lax.axis_index("s")
    wid = sid * NC + cid
    slab0 = wid * ROWS_PER_W

    in_bufs = (in0, in1)
    out_bufs = (out0, out1)
    in_sems = (si0, si1)
    out_sems = (so0, so1)

    def start_in(j, b):
        slab = slab0 + j // BLOCKS_PER_SLAB
        h0 = (j % BLOCKS_PER_SLAB) * BH
        return pltpu.async_copy(
            x_hbm.at[slab, pl.ds(h0, BH), :], in_bufs[b], in_sems[b])

    def wait_in(b):
        pltpu.make_async_copy(
            x_hbm.at[0, pl.ds(0, BH), :], in_bufs[b], in_sems[b]).wait()

    def start_out(j, b):
        slab = slab0 + j // BLOCKS_PER_SLAB
        h0 = (j % BLOCKS_PER_SLAB) * BH
        return pltpu.async_copy(
            out_bufs[b], out_hbm.at[slab, pl.ds(h0, BH), :], out_sems[b])

    def wait_out(b):
        pltpu.make_async_copy(
            out_bufs[b], out_hbm.at[0, pl.ds(0, BH), :], out_sems[b]).wait()

    start_in(0, 0)
    start_in(1, 1)
    pltpu.sync_copy(a_hbm, a_v)
    pltpu.sync_copy(b_hbm, b_v)
    pltpu.sync_copy(prm_hbm, prm_v)
    pv = prm_v[pl.ds(0, LANES)]
    inv_step = pv[0]
    toff = pv[1]

    def pair_body(jj, carry):
        for b in range(2):
            j = jj * 2 + b
            in_buf = in_bufs[b]
            out_buf = out_bufs[b]

            slab = slab0 + j // BLOCKS_PER_SLAB
            clo = lax.rem(slab, C_) * TAB
            chi = clo + (TAB - 1)
            boff = toff + (clo + 1).astype(jnp.float32)

            wait_in(b)

            @pl.when(jj > 0)
            def _():
                wait_out(b)

            @plsc.parallel_loop(0, BH, 1, unroll=1)
            def _(h):
                for w in range(WREGS):
                    x = in_buf[h, pl.ds(w * LANES, LANES)]
                    t = x * inv_step + boff
                    ti = t.astype(jnp.int32)
                    ki = jnp.clip(ti, clo, chi)
                    a = plsc.load_gather(a_v, [ki])
                    bb = plsc.load_gather(b_v, [ki])
                    out_buf[h, pl.ds(w * LANES, LANES)] = a * x + bb

            start_out(j, b)

            @pl.when(j + 2 < NBLOCK)
            def _():
                start_in(j + 2, b)

        return carry

    lax.fori_loop(0, NBLOCK // 2, pair_body, 0)
    wait_out(0)
    wait_out(1)


def kernel(input, ps, qs):
    qs = qs.astype(jnp.float32)
    ps = ps.astype(jnp.float32)
    step = ps[1] - ps[0]
    inv_step = 1.0 / step
    s_mid = (qs[1:] - qs[:-1]) * inv_step
    b_mid = qs[:-1] - s_mid * ps[:-1, None]
    ones = jnp.ones((1, C_), jnp.float32)
    a_tab = jnp.concatenate([ones, s_mid, ones], axis=0)
    b_tab = jnp.concatenate(
        [(qs[0] - ps[0])[None, :], b_mid, (qs[-1] - ps[-1])[None, :]], axis=0
    )
    a_flat = a_tab.T.reshape(-1)
    b_flat = b_tab.T.reshape(-1)
    prm = jnp.zeros((LANES,), jnp.float32)
    prm = prm.at[0].set(inv_step).at[1].set(-ps[0] * inv_step)

    out3 = _sc_pwl(input.reshape(ROWS, H_, W_), a_flat, b_flat, prm)
    return out3.reshape(input.shape)
